# baseline (speedup 1.0000x reference)
import functools

import jax
import jax.numpy as jnp
from jax import lax
from jax.experimental import pallas as pl
from jax.experimental.pallas import tpu as pltpu
from jax.experimental.pallas import tpu_sc as plsc

N = 10000
NP = 10112
E = 160000
D = 256
H = 128
NT = 16
CH = 64
NCHT = 160
EPT = NCHT * CH
EP = NT * EPT
HB = EPT // 2
NCH_H = HB // CH
RING = 3
RPT = NP // NT
NSTEP = 6
F32 = jnp.float32
I32 = jnp.int32

_mesh = plsc.VectorSubcoreMesh(core_axis_name="c", subcore_axis_name="s")


@functools.partial(
    pl.kernel,
    mesh=_mesh,
    out_type=[
        jax.ShapeDtypeStruct((10 * NP, H), F32),
    ],
    scratch_types=(
        [pltpu.VMEM_SHARED((NP, H), F32)]
        + [pltpu.VMEM((CH, H), F32) for _ in range(RING)]
        + [pltpu.VMEM((HB,), I32),
           pltpu.VMEM((NCH_H, CH), I32),
           pltpu.VMEM((HB,), F32)]
        + [pltpu.SemaphoreType.DMA for _ in range(2 * RING + 1)]
    ),
)
def _dchl(xs, zin, pk_h, vals_h, xbuf, acc, *rest):
    rbuf = rest[0:RING]
    pkh = rest[RING]
    rh2d = rest[RING + 1]
    vh = rest[RING + 2]
    gsem = rest[RING + 3:RING + 3 + RING]
    ssem = rest[RING + 3 + RING:RING + 3 + 2 * RING]
    bsem = rest[RING + 3 + 2 * RING]

    c = lax.axis_index("c")
    s = lax.axis_index("s")
    coff = c * NP
    rbase = s * RPT
    ebase = s * EPT

    pltpu.sync_copy(xs.at[pl.ds(coff + rbase, RPT)],
                    xbuf.at[pl.ds(2 * NP + coff + rbase, RPT)])
    plsc.subcore_barrier()

    def step_body(k, carry):
        layer = k // 2
        is_s = k % 2
        xl_off = 2 * NP * (1 + layer)
        table_off = jnp.where(is_s == 0, xl_off, 0) + coff
        wb_off = jnp.where(is_s == 0, 0, xl_off + 2 * NP) + coff
        edge_off = is_s * EP

        @pl.when(is_s == 0)
        def _():
            pltpu.sync_copy(zin.at[pl.ds(rbase, RPT)],
                            acc.at[pl.ds(rbase, RPT)])

        @pl.when(is_s == 1)
        def _():
            pltpu.sync_copy(xbuf.at[pl.ds(xl_off + coff + rbase, RPT)],
                            acc.at[pl.ds(rbase, RPT)])
        plsc.subcore_barrier()

        def gstart(p, b):
            pltpu.make_async_copy(
                xbuf.at[pkh.at[pl.ds(p * CH, CH)]], rbuf[b], gsem[b]).start()

        def gwait(p, b):
            pltpu.make_async_copy(
                xbuf.at[pkh.at[pl.ds(p * CH, CH)]], rbuf[b], gsem[b]).wait()

        def consume(p, b):
            gwait(p, b)
            rb = rbuf[b]

            def scale_body(g, carry2):
                v16 = vh[pl.ds(p * CH + 16 * g, 16)]
                for e in range(16):
                    vb = jnp.broadcast_to(v16[e], (16,))
                    r = g * 16 + e
                    for kk in range(H // 16):
                        rb[r, pl.ds(kk * 16, 16)] = (
                            rb[r, pl.ds(kk * 16, 16)] * vb)
                return carry2
            lax.fori_loop(0, CH // 16, scale_body, 0)
            pltpu.make_async_copy(rbuf[b],
                                  acc.at[rh2d.at[p]], ssem[b]).start(add=True)

        def scat_wait(p, b):
            pltpu.make_async_copy(rbuf[b], acc.at[rh2d.at[p]], ssem[b]).wait()

        for hb in range(2):
            hbase = edge_off + ebase + hb * HB
            cp = pltpu.make_async_copy(pk_h.at[pl.ds(hbase, HB)], pkh, bsem)
            cv = pltpu.make_async_copy(vals_h.at[pl.ds(hbase, HB)], vh, bsem)
            cp.start(); cv.start()
            cp.wait(); cv.wait()

            def unpack_body(u, carry2):
                pk16 = pkh[pl.ds(16 * u, 16)]
                rh2d[u // (CH // 16), pl.ds((u % (CH // 16)) * 16, 16)] = (
                    lax.shift_right_logical(pk16, 14))
                pkh[pl.ds(16 * u, 16)] = (pk16 & 16383) + table_off
                return carry2
            lax.fori_loop(0, HB // 16, unpack_body, 0)

            gstart(0, 0)
            gstart(1, 1)
            consume(0, 0)
            gstart(2, 2)

            def steady(g, carry2):
                p0 = 1 + RING * g
                for b in range(RING):
                    p = p0 + b
                    consume(p, (1 + b) % RING)
                    scat_wait(p - 1, b)
                    gstart(p + 2, b)
                return carry2
            lax.fori_loop(0, 25, steady, 0)

            consume(76, 76 % RING)
            scat_wait(75, 75 % RING)
            gstart(78, 78 % RING)
            consume(77, 77 % RING)
            scat_wait(76, 76 % RING)
            gstart(79, 79 % RING)
            consume(78, 78 % RING)
            scat_wait(77, 77 % RING)
            consume(79, 79 % RING)
            scat_wait(78, 78 % RING)
            scat_wait(79, 79 % RING)
        plsc.subcore_barrier()

        pltpu.sync_copy(acc.at[pl.ds(rbase, RPT)],
                        xbuf.at[pl.ds(wb_off + rbase, RPT)])
        plsc.subcore_barrier()
        return carry

    lax.fori_loop(0, NSTEP, step_body, 0)


_BN = 400


def _mean_body(x0, a0, a1, b0, b1, c0, c1, o):
    cat = jnp.concatenate
    o[...] = (x0[...]
              + cat([a0[0], a1[0]], axis=1)
              + cat([b0[0], b1[0]], axis=1)
              + cat([c0[0], c1[0]], axis=1)) * 0.25


def _mean_tc(x0, xbuf):
    xb = xbuf.reshape(10, NP, H)

    def spec(p):
        return pl.BlockSpec((1, _BN, H), lambda i, p=p: (p, i, 0))

    return pl.pallas_call(
        _mean_body,
        grid=(N // _BN,),
        in_specs=[pl.BlockSpec((_BN, D), lambda i: (i, 0)),
                  spec(4), spec(5), spec(6), spec(7), spec(8), spec(9)],
        out_specs=pl.BlockSpec((_BN, D), lambda i: (i, 0)),
        out_shape=jax.ShapeDtypeStruct((N, D), F32),
    )(x0, xb, xb, xb, xb, xb, xb)


def _pad(arr, dtype):
    return jnp.concatenate([arr, jnp.zeros((EP - E,), dtype)])


def kernel(pois_embs, src_indices, src_values, tar_indices, tar_values):
    xh = pois_embs.reshape(N, 2, H).transpose(1, 0, 2)
    xs = jnp.pad(xh, ((0, 0), (0, NP - N), (0, 0))).reshape(2 * NP, H)
    zin = jnp.zeros((NP, H), F32)
    pk_h = jnp.concatenate(
        [_pad(tar_indices[0] * 16384 + tar_indices[1], I32),
         _pad(src_indices[0] * 16384 + src_indices[1], I32)])
    vals_h = jnp.concatenate([_pad(tar_values, F32), _pad(src_values, F32)])
    res = _dchl(xs, zin, pk_h, vals_h)
    xbuf = res[0] if isinstance(res, (list, tuple)) else res
    return _mean_tc(pois_embs, xbuf)


# speedup vs baseline: 1.0942x; 1.0942x over previous
import functools

import jax
import jax.numpy as jnp
from jax import lax
from jax.experimental import pallas as pl
from jax.experimental.pallas import tpu as pltpu
from jax.experimental.pallas import tpu_sc as plsc

N = 10000
NP = 10240
E = 160000
D = 256
H = 128
NT = 16
CH = 80
EPT = 10240
EP = NT * EPT
HB = EPT // 2
NCH_H = HB // CH
RING = 3
RPT = NP // NT
NSTEP = 6
F32 = jnp.float32
I32 = jnp.int32

_mesh = plsc.VectorSubcoreMesh(core_axis_name="c", subcore_axis_name="s")


@functools.partial(
    pl.kernel,
    mesh=_mesh,
    out_type=[
        jax.ShapeDtypeStruct((10 * NP, H), F32),
    ],
    scratch_types=(
        [pltpu.VMEM_SHARED((NP, H), F32)]
        + [pltpu.VMEM((CH, H), F32) for _ in range(RING)]
        + [pltpu.VMEM((HB,), I32),
           pltpu.VMEM((NCH_H, CH), I32),
           pltpu.VMEM((HB,), F32)]
        + [pltpu.SemaphoreType.DMA for _ in range(2 * RING + 1)]
    ),
)
def _dchl(xs, zin, pk_h, vals_h, xbuf, acc, *rest):
    rbuf = rest[0:RING]
    pkh = rest[RING]
    rh2d = rest[RING + 1]
    vh = rest[RING + 2]
    gsem = rest[RING + 3:2 * RING + 3]
    ssem = rest[2 * RING + 3:3 * RING + 3]
    bsem = rest[3 * RING + 3]

    c = lax.axis_index("c")
    s = lax.axis_index("s")
    coff = c * NP
    rbase = s * RPT
    ebase = s * EPT

    pltpu.sync_copy(xs.at[pl.ds(coff + rbase, RPT)],
                    xbuf.at[pl.ds(2 * NP + coff + rbase, RPT)])
    plsc.subcore_barrier()

    def step_body(k, carry):
        layer = k // 2
        is_s = k % 2
        xl_off = 2 * NP * (1 + layer)
        table_off = jnp.where(is_s == 0, xl_off, 0) + coff
        wb_off = jnp.where(is_s == 0, 0, xl_off + 2 * NP) + coff
        edge_off = is_s * EP

        @pl.when(is_s == 0)
        def _():
            pltpu.sync_copy(zin.at[pl.ds(rbase, RPT)],
                            acc.at[pl.ds(rbase, RPT)])

        @pl.when(is_s == 1)
        def _():
            pltpu.sync_copy(xbuf.at[pl.ds(xl_off + coff + rbase, RPT)],
                            acc.at[pl.ds(rbase, RPT)])
        plsc.subcore_barrier()

        def gstart(p, b):
            pltpu.make_async_copy(
                xbuf.at[pkh.at[pl.ds(p * CH, CH)]], rbuf[b], gsem[b]).start()

        def consume(p, b):
            pltpu.make_async_copy(
                xbuf.at[pkh.at[pl.ds(p * CH, CH)]], rbuf[b], gsem[b]).wait()
            rb = rbuf[b]

            def scale_body(g, carry2):
                v16 = vh[pl.ds(p * CH + 16 * g, 16)]
                for e in range(16):
                    vb = jnp.broadcast_to(v16[e], (16,))
                    r = g * 16 + e
                    for kk in range(H // 16):
                        rb[r, pl.ds(kk * 16, 16)] = (
                            rb[r, pl.ds(kk * 16, 16)] * vb)
                return carry2
            lax.fori_loop(0, CH // 16, scale_body, 0)
            pltpu.make_async_copy(rbuf[b],
                                  acc.at[rh2d.at[p]], ssem[b]).start(add=True)

        def scat_wait(p, b):
            pltpu.make_async_copy(rbuf[b], acc.at[rh2d.at[p]], ssem[b]).wait()

        for hb in range(2):
            hbase = edge_off + ebase + hb * HB
            cp = pltpu.make_async_copy(pk_h.at[pl.ds(hbase, HB)], pkh, bsem)
            cv = pltpu.make_async_copy(vals_h.at[pl.ds(hbase, HB)], vh, bsem)
            cp.start(); cv.start()
            cp.wait(); cv.wait()

            def unpack_body(u, carry2):
                pk16 = pkh[pl.ds(16 * u, 16)]
                rh2d[u // (CH // 16), pl.ds((u % (CH // 16)) * 16, 16)] = (
                    lax.shift_right_logical(pk16, 14))
                pkh[pl.ds(16 * u, 16)] = (pk16 & 16383) + table_off
                return carry2
            lax.fori_loop(0, HB // 16, unpack_body, 0)

            gstart(0, 0)
            gstart(1, 1)
            consume(0, 0)
            gstart(2, 2)

            def steady(g, carry2):
                p0 = 1 + RING * g
                for b in range(RING):
                    p = p0 + b
                    consume(p, (1 + b) % RING)
                    scat_wait(p - 1, b)
                    gstart(p + 2, b)
                return carry2
            lax.fori_loop(0, (NCH_H - 4) // RING, steady, 0)

            consume(61, 61 % RING)
            scat_wait(60, 60 % RING)
            gstart(63, 63 % RING)
            consume(62, 62 % RING)
            scat_wait(61, 61 % RING)
            consume(63, 63 % RING)
            scat_wait(62, 62 % RING)
            scat_wait(63, 63 % RING)
        plsc.subcore_barrier()

        pltpu.sync_copy(acc.at[pl.ds(rbase, RPT)],
                        xbuf.at[pl.ds(wb_off + rbase, RPT)])
        plsc.subcore_barrier()
        return carry

    lax.fori_loop(0, NSTEP, step_body, 0)


_BN = 400


def _mean_body(x0, a0, a1, b0, b1, c0, c1, o):
    cat = jnp.concatenate
    o[...] = (x0[...]
              + cat([a0[0], a1[0]], axis=1)
              + cat([b0[0], b1[0]], axis=1)
              + cat([c0[0], c1[0]], axis=1)) * 0.25


def _mean_tc(x0, xbuf):
    xb = xbuf.reshape(10, NP, H)

    def spec(p):
        return pl.BlockSpec((1, _BN, H), lambda i, p=p: (p, i, 0))

    return pl.pallas_call(
        _mean_body,
        grid=(N // _BN,),
        in_specs=[pl.BlockSpec((_BN, D), lambda i: (i, 0)),
                  spec(4), spec(5), spec(6), spec(7), spec(8), spec(9)],
        out_specs=pl.BlockSpec((_BN, D), lambda i: (i, 0)),
        out_shape=jax.ShapeDtypeStruct((N, D), F32),
    )(x0, xb, xb, xb, xb, xb, xb)


def _pad(arr, dtype):
    return jnp.concatenate([arr, jnp.zeros((EP - E,), dtype)])


def kernel(pois_embs, src_indices, src_values, tar_indices, tar_values):
    xh = pois_embs.reshape(N, 2, H).transpose(1, 0, 2)
    xs = jnp.pad(xh, ((0, 0), (0, NP - N), (0, 0))).reshape(2 * NP, H)
    zin = jnp.zeros((NP, H), F32)
    pk_h = jnp.concatenate(
        [_pad(tar_indices[0] * 16384 + tar_indices[1], I32),
         _pad(src_indices[0] * 16384 + src_indices[1], I32)])
    vals_h = jnp.concatenate([_pad(tar_values, F32), _pad(src_values, F32)])
    res = _dchl(xs, zin, pk_h, vals_h)
    xbuf = res[0] if isinstance(res, (list, tuple)) else res
    return _mean_tc(pois_embs, xbuf)


# speedup vs baseline: 1.1004x; 1.0057x over previous
import functools

import jax
import jax.numpy as jnp
from jax import lax
from jax.experimental import pallas as pl
from jax.experimental.pallas import tpu as pltpu
from jax.experimental.pallas import tpu_sc as plsc

N = 10000
NP = 10240
E = 160000
D = 256
H = 128
NT = 16
CH = 64
EPT = 10240
EP = NT * EPT
HB = EPT // 4
NCH_H = HB // CH
RING = 4
RPT = NP // NT
NSTEP = 6
F32 = jnp.float32
I32 = jnp.int32

_mesh = plsc.VectorSubcoreMesh(core_axis_name="c", subcore_axis_name="s")


@functools.partial(
    pl.kernel,
    mesh=_mesh,
    out_type=[
        jax.ShapeDtypeStruct((10 * NP, H), F32),
    ],
    scratch_types=(
        [pltpu.VMEM_SHARED((NP, H), F32)]
        + [pltpu.VMEM((CH, H), F32) for _ in range(RING)]
        + [pltpu.VMEM((HB,), I32),
           pltpu.VMEM((NCH_H, CH), I32),
           pltpu.VMEM((HB,), F32)]
        + [pltpu.SemaphoreType.DMA for _ in range(2 * RING + 1)]
    ),
)
def _dchl(xs, zin, pk_h, vals_h, xbuf, acc, *rest):
    rbuf = rest[0:RING]
    pkh = rest[RING]
    rh2d = rest[RING + 1]
    vh = rest[RING + 2]
    gsem = rest[RING + 3:2 * RING + 3]
    ssem = rest[2 * RING + 3:3 * RING + 3]
    bsem = rest[3 * RING + 3]

    c = lax.axis_index("c")
    s = lax.axis_index("s")
    coff = c * NP
    rbase = s * RPT
    ebase = s * EPT

    pltpu.sync_copy(xs.at[pl.ds(coff + rbase, RPT)],
                    xbuf.at[pl.ds(2 * NP + coff + rbase, RPT)])
    plsc.subcore_barrier()

    def step_body(k, carry):
        layer = k // 2
        is_s = k % 2
        xl_off = 2 * NP * (1 + layer)
        table_off = jnp.where(is_s == 0, xl_off, 0) + coff
        wb_off = jnp.where(is_s == 0, 0, xl_off + 2 * NP) + coff
        edge_off = is_s * EP

        @pl.when(is_s == 0)
        def _():
            pltpu.sync_copy(zin.at[pl.ds(rbase, RPT)],
                            acc.at[pl.ds(rbase, RPT)])

        @pl.when(is_s == 1)
        def _():
            pltpu.sync_copy(xbuf.at[pl.ds(xl_off + coff + rbase, RPT)],
                            acc.at[pl.ds(rbase, RPT)])
        plsc.subcore_barrier()

        def gstart(p, b):
            pltpu.make_async_copy(
                xbuf.at[pkh.at[pl.ds(p * CH, CH)]], rbuf[b], gsem[b]).start()

        def consume(p, b):
            pltpu.make_async_copy(
                xbuf.at[pkh.at[pl.ds(p * CH, CH)]], rbuf[b], gsem[b]).wait()
            rb = rbuf[b]

            def scale_body(g, carry2):
                v16 = vh[pl.ds(p * CH + 16 * g, 16)]
                for e in range(16):
                    vb = jnp.broadcast_to(v16[e], (16,))
                    r = g * 16 + e
                    for kk in range(H // 16):
                        rb[r, pl.ds(kk * 16, 16)] = (
                            rb[r, pl.ds(kk * 16, 16)] * vb)
                return carry2
            lax.fori_loop(0, CH // 16, scale_body, 0)
            pltpu.make_async_copy(rbuf[b],
                                  acc.at[rh2d.at[p]], ssem[b]).start(add=True)

        def scat_wait(p, b):
            pltpu.make_async_copy(rbuf[b], acc.at[rh2d.at[p]], ssem[b]).wait()

        for hb in range(4):
            hbase = edge_off + ebase + hb * HB
            cp = pltpu.make_async_copy(pk_h.at[pl.ds(hbase, HB)], pkh, bsem)
            cv = pltpu.make_async_copy(vals_h.at[pl.ds(hbase, HB)], vh, bsem)
            cp.start(); cv.start()
            cp.wait(); cv.wait()

            def unpack_body(u, carry2):
                pk16 = pkh[pl.ds(16 * u, 16)]
                rh2d[u // (CH // 16), pl.ds((u % (CH // 16)) * 16, 16)] = (
                    lax.shift_right_logical(pk16, 14))
                pkh[pl.ds(16 * u, 16)] = (pk16 & 16383) + table_off
                return carry2
            lax.fori_loop(0, HB // 16, unpack_body, 0)

            gstart(0, 0)
            gstart(1, 1)
            gstart(2, 2)
            consume(0, 0)
            gstart(3, 3)

            def steady(g, carry2):
                p0 = 1 + RING * g
                for b in range(RING):
                    p = p0 + b
                    consume(p, (1 + b) % RING)
                    scat_wait(p - 1, b)
                    gstart(p + 3, b)
                return carry2
            lax.fori_loop(0, (NCH_H - 4) // RING, steady, 0)

            consume(37, 37 % RING)
            scat_wait(36, 36 % RING)
            consume(38, 38 % RING)
            scat_wait(37, 37 % RING)
            consume(39, 39 % RING)
            scat_wait(38, 38 % RING)
            scat_wait(39, 39 % RING)
        plsc.subcore_barrier()

        pltpu.sync_copy(acc.at[pl.ds(rbase, RPT)],
                        xbuf.at[pl.ds(wb_off + rbase, RPT)])
        plsc.subcore_barrier()
        return carry

    lax.fori_loop(0, NSTEP, step_body, 0)


_BN = 400


def _mean_body(x0, a0, a1, b0, b1, c0, c1, o):
    cat = jnp.concatenate
    o[...] = (x0[...]
              + cat([a0[0], a1[0]], axis=1)
              + cat([b0[0], b1[0]], axis=1)
              + cat([c0[0], c1[0]], axis=1)) * 0.25


def _mean_tc(x0, xbuf):
    xb = xbuf.reshape(10, NP, H)

    def spec(p):
        return pl.BlockSpec((1, _BN, H), lambda i, p=p: (p, i, 0))

    return pl.pallas_call(
        _mean_body,
        grid=(N // _BN,),
        in_specs=[pl.BlockSpec((_BN, D), lambda i: (i, 0)),
                  spec(4), spec(5), spec(6), spec(7), spec(8), spec(9)],
        out_specs=pl.BlockSpec((_BN, D), lambda i: (i, 0)),
        out_shape=jax.ShapeDtypeStruct((N, D), F32),
    )(x0, xb, xb, xb, xb, xb, xb)


def _pad(arr, dtype):
    return jnp.concatenate([arr, jnp.zeros((EP - E,), dtype)])


def kernel(pois_embs, src_indices, src_values, tar_indices, tar_values):
    xh = pois_embs.reshape(N, 2, H).transpose(1, 0, 2)
    xs = jnp.pad(xh, ((0, 0), (0, NP - N), (0, 0))).reshape(2 * NP, H)
    zin = jnp.zeros((NP, H), F32)
    pk_h = jnp.concatenate(
        [_pad(tar_indices[0] * 16384 + tar_indices[1], I32),
         _pad(src_indices[0] * 16384 + src_indices[1], I32)])
    vals_h = jnp.concatenate([_pad(tar_values, F32), _pad(src_values, F32)])
    res = _dchl(xs, zin, pk_h, vals_h)
    xbuf = res[0] if isinstance(res, (list, tuple)) else res
    return _mean_tc(pois_embs, xbuf)
